# baseline (speedup 1.0000x reference)
import functools

import jax
import jax.numpy as jnp
from jax import lax
from jax.experimental import pallas as pl
from jax.experimental.pallas import tpu as pltpu
from jax.experimental.pallas import tpu_sc as plsc

_VOCAB = 1000002
_EMB = 64
_DIM = 256
_NCLS = 2
_B = 4096
_L = 200
_EPS = 1e-5

_NC = 2
_NS = 16
_NW = _NC * _NS
_BPW = _B // _NW
_LC = 104
_NC2 = 2
_LP = _NC2 * _LC


def _pool_body(x_hbm, table_hbm, out_hbm, idx_v, rows_v, out_v, sem):
    wid = lax.axis_index("s") * _NC + lax.axis_index("c")
    base = wid * _BPW

    def row_fn(i, _):
        b = base + i
        pltpu.sync_copy(x_hbm.at[b], idx_v)
        cps = []
        for j in range(_NC2):
            cps.append(
                pltpu.async_copy(
                    table_hbm.at[idx_v.at[j]],
                    rows_v.at[pl.ds(j * _LC, _LC)],
                    sem,
                )
            )
        for cp in cps:
            cp.wait()

        zero = jnp.zeros((16,), jnp.float32)

        def acc_fn(r, carry):
            a0, a1, a2, a3 = carry
            a0 = a0 + rows_v[r, pl.ds(0, 16)]
            a1 = a1 + rows_v[r, pl.ds(16, 16)]
            a2 = a2 + rows_v[r, pl.ds(32, 16)]
            a3 = a3 + rows_v[r, pl.ds(48, 16)]
            return a0, a1, a2, a3

        a0, a1, a2, a3 = lax.fori_loop(
            0, _L, acc_fn, (zero, zero, zero, zero), unroll=2
        )
        scale = jnp.float32(1.0 / _L)
        out_v[pl.ds(0, 16)] = a0 * scale
        out_v[pl.ds(16, 16)] = a1 * scale
        out_v[pl.ds(32, 16)] = a2 * scale
        out_v[pl.ds(48, 16)] = a3 * scale
        pltpu.sync_copy(out_v, out_hbm.at[b])
        return ()

    lax.fori_loop(0, _BPW, row_fn, ())


@functools.partial(jax.jit, static_argnames=())
def _pool_mean(x2, table):
    mesh = plsc.VectorSubcoreMesh(core_axis_name="c", subcore_axis_name="s")
    k = pl.kernel(
        _pool_body,
        out_type=jax.ShapeDtypeStruct((_B, _EMB), jnp.float32),
        mesh=mesh,
        scratch_types=[
            pltpu.VMEM((_NC2, _LC), jnp.int32),
            pltpu.VMEM((_LP, _EMB), jnp.float32),
            pltpu.VMEM((_EMB,), jnp.float32),
            pltpu.SemaphoreType.DMA,
        ],
        compiler_params=pltpu.CompilerParams(use_tc_tiling_on_sc=False),
    )
    return k(x2, table)


_VB = 4096
_VGRID = -(-_VOCAB // _VB)
_VPAD = _VGRID * _VB


def _detile_body(tT_ref, out_ref):
    blk = tT_ref[...]
    left = blk[:, : _VB // 2].T
    right = blk[:, _VB // 2 :].T
    out_ref[...] = jnp.concatenate([left, right], axis=1)


def _detile(tT):
    return pl.pallas_call(
        _detile_body,
        grid=(_VGRID,),
        in_specs=[pl.BlockSpec((_EMB, _VB), lambda i: (0, i))],
        out_specs=pl.BlockSpec((_VB // 2, 2 * _EMB), lambda i: (i, 0)),
        out_shape=jax.ShapeDtypeStruct((_VPAD // 2, 2 * _EMB), jnp.float32),
    )(tT)


def _mlp_body(h_ref, g1_ref, b1_ref, W1_ref, bf1_ref, g2_ref, b2_ref,
              W2_ref, bf2_ref, out_ref):
    h = h_ref[...]
    mu1 = jnp.mean(h, axis=0, keepdims=True)
    d1 = h - mu1
    var1 = jnp.mean(d1 * d1, axis=0, keepdims=True)
    h = d1 * lax.rsqrt(var1 + _EPS) * g1_ref[...] + b1_ref[...]
    h = lax.dot_general(h, W1_ref[...], (((1,), (1,)), ((), ())),
                        preferred_element_type=jnp.float32) + bf1_ref[...]
    mu2 = jnp.mean(h, axis=0, keepdims=True)
    d2 = h - mu2
    var2 = jnp.mean(d2 * d2, axis=0, keepdims=True)
    h = d2 * lax.rsqrt(var2 + _EPS) * g2_ref[...] + b2_ref[...]
    out = lax.dot_general(h, W2_ref[...], (((1,), (1,)), ((), ())),
                          preferred_element_type=jnp.float32) + bf2_ref[...]
    out_ref[...] = out


def _mlp(h, g1, b1, W1, bf1, g2, b2, W2, bf2):
    return pl.pallas_call(
        _mlp_body,
        out_shape=jax.ShapeDtypeStruct((_B, _NCLS), jnp.float32),
    )(h, g1.reshape(1, _EMB), b1.reshape(1, _EMB), W1, bf1.reshape(1, _DIM),
      g2.reshape(1, _DIM), b2.reshape(1, _DIM), W2, bf2.reshape(1, _NCLS))


def kernel(x, table, g1, b1, W1, bf1, g2, b2, W2, bf2):
    table_lin = _detile(table.T).reshape(_VPAD, _EMB)
    h = _VB // 2
    xr = (x // _VB) * _VB + 2 * (x % h) + (x % _VB) // h
    xp = jnp.pad(xr, ((0, 0), (0, _LP - _L))).reshape(_B, _NC2, _LC)
    h = _pool_mean(xp, table_lin)
    return _mlp(h, g1, b1, W1, bf1, g2, b2, W2, bf2)


# speedup vs baseline: 1.7102x; 1.7102x over previous
import functools

import jax
import jax.numpy as jnp
from jax import lax
from jax.experimental import pallas as pl
from jax.experimental.pallas import tpu as pltpu
from jax.experimental.pallas import tpu_sc as plsc

_VOCAB = 1000002
_EMB = 64
_DIM = 256
_NCLS = 2
_B = 4096
_L = 200
_EPS = 1e-5

_NC = 2
_NS = 16
_NW = _NC * _NS
_BPW = _B // _NW
_CHUNKS = ((0, 104), (104, 96))


def _pool_body(x_hbm, table_hbm, out_hbm, idx_v, rows_v, out_v, sem):
    wid = lax.axis_index("s") * _NC + lax.axis_index("c")
    base = wid * _BPW

    def row_fn(i, _):
        b = base + i
        pltpu.sync_copy(x_hbm.at[b], idx_v)
        cps = []
        for off, n in _CHUNKS:
            cps.append(
                pltpu.async_copy(
                    table_hbm.at[idx_v.at[pl.ds(off, n)]],
                    rows_v.at[pl.ds(off, n)],
                    sem,
                )
            )
        for cp in cps:
            cp.wait()

        zero = jnp.zeros((16,), jnp.float32)

        def acc_fn(r, carry):
            a0, a1, a2, a3 = carry
            a0 = a0 + rows_v[r, pl.ds(0, 16)]
            a1 = a1 + rows_v[r, pl.ds(16, 16)]
            a2 = a2 + rows_v[r, pl.ds(32, 16)]
            a3 = a3 + rows_v[r, pl.ds(48, 16)]
            return a0, a1, a2, a3

        a0, a1, a2, a3 = lax.fori_loop(
            0, _L, acc_fn, (zero, zero, zero, zero), unroll=2
        )
        scale = jnp.float32(1.0 / _L)
        out_v[pl.ds(0, 16)] = a0 * scale
        out_v[pl.ds(16, 16)] = a1 * scale
        out_v[pl.ds(32, 16)] = a2 * scale
        out_v[pl.ds(48, 16)] = a3 * scale
        pltpu.sync_copy(out_v, out_hbm.at[b])
        return ()

    lax.fori_loop(0, _BPW, row_fn, ())


@functools.partial(jax.jit, static_argnames=())
def _pool_mean(x2, table):
    mesh = plsc.VectorSubcoreMesh(core_axis_name="c", subcore_axis_name="s")
    k = pl.kernel(
        _pool_body,
        out_type=jax.ShapeDtypeStruct((_B, _EMB), jnp.float32),
        mesh=mesh,
        scratch_types=[
            pltpu.VMEM((_L,), jnp.int32),
            pltpu.VMEM((_L, _EMB), jnp.float32),
            pltpu.VMEM((_EMB,), jnp.float32),
            pltpu.SemaphoreType.DMA,
        ],
        compiler_params=pltpu.CompilerParams(use_tc_tiling_on_sc=False),
    )
    return k(x2, table)


_VB = 4096
_VGRID = -(-_VOCAB // _VB)
_VPAD = _VGRID * _VB


def _detile_body(tT_ref, out_ref):
    blk = tT_ref[...]
    left = blk[:, : _VB // 2].T
    right = blk[:, _VB // 2 :].T
    out_ref[...] = jnp.concatenate([left, right], axis=1)


def _detile(tT):
    return pl.pallas_call(
        _detile_body,
        grid=(_VGRID,),
        in_specs=[pl.BlockSpec((_EMB, _VB), lambda i: (0, i))],
        out_specs=pl.BlockSpec((_VB // 2, 2 * _EMB), lambda i: (i, 0)),
        out_shape=jax.ShapeDtypeStruct((_VPAD // 2, 2 * _EMB), jnp.float32),
    )(tT)


def _mlp_body(h_ref, g1_ref, b1_ref, W1_ref, bf1_ref, g2_ref, b2_ref,
              W2_ref, bf2_ref, out_ref):
    h = h_ref[...]
    mu1 = jnp.mean(h, axis=0, keepdims=True)
    d1 = h - mu1
    var1 = jnp.mean(d1 * d1, axis=0, keepdims=True)
    h = d1 * lax.rsqrt(var1 + _EPS) * g1_ref[...] + b1_ref[...]
    h = lax.dot_general(h, W1_ref[...], (((1,), (1,)), ((), ())),
                        preferred_element_type=jnp.float32) + bf1_ref[...]
    mu2 = jnp.mean(h, axis=0, keepdims=True)
    d2 = h - mu2
    var2 = jnp.mean(d2 * d2, axis=0, keepdims=True)
    h = d2 * lax.rsqrt(var2 + _EPS) * g2_ref[...] + b2_ref[...]
    out = lax.dot_general(h, W2_ref[...], (((1,), (1,)), ((), ())),
                          preferred_element_type=jnp.float32) + bf2_ref[...]
    out_ref[...] = out


def _mlp(h, g1, b1, W1, bf1, g2, b2, W2, bf2):
    return pl.pallas_call(
        _mlp_body,
        out_shape=jax.ShapeDtypeStruct((_B, _NCLS), jnp.float32),
    )(h, g1.reshape(1, _EMB), b1.reshape(1, _EMB), W1, bf1.reshape(1, _DIM),
      g2.reshape(1, _DIM), b2.reshape(1, _DIM), W2, bf2.reshape(1, _NCLS))


def kernel(x, table, g1, b1, W1, bf1, g2, b2, W2, bf2):
    table_lin = _detile(table.T).reshape(_VPAD, _EMB)
    xr = (x & ~0xFFF) | ((x & 0x7FF) << 1) | ((x >> 11) & 1)
    h = _pool_mean(xr, table_lin)
    return _mlp(h, g1, b1, W1, bf1, g2, b2, W2, bf2)


# speedup vs baseline: 2.2733x; 1.3293x over previous
import functools

import jax
import jax.numpy as jnp
from jax import lax
from jax.experimental import pallas as pl
from jax.experimental.pallas import tpu as pltpu
from jax.experimental.pallas import tpu_sc as plsc

_VOCAB = 1000002
_EMB = 64
_DIM = 256
_NCLS = 2
_B = 4096
_L = 200
_EPS = 1e-5

_NC = 2
_NS = 16
_NW = _NC * _NS
_BPW = _B // _NW
_CHUNKS = ((0, 104), (104, 96))


def _pool_body(x_hbm, table_hbm, out_hbm, idx_v, rows_v, out_v,
               isem0, isem1, sem0, sem1):
    wid = lax.axis_index("s") * _NC + lax.axis_index("c")
    base = wid * _BPW
    sems = (sem0, sem1)
    isems = (isem0, isem1)

    def start_idx(i, buf):
        pltpu.async_copy(x_hbm.at[i], idx_v.at[pl.ds(buf * _L, _L)],
                         isems[buf])

    def wait_idx(buf):
        pltpu.make_async_copy(x_hbm.at[0], idx_v.at[pl.ds(buf * _L, _L)],
                              isems[buf]).wait()

    def start(buf):
        for off, n in _CHUNKS:
            pltpu.async_copy(
                table_hbm.at[idx_v.at[pl.ds(buf * _L + off, n)]],
                rows_v.at[pl.ds(buf * _L + off, n)],
                sems[buf],
            )

    def drain(buf):
        for off, n in _CHUNKS:
            pltpu.make_async_copy(
                table_hbm.at[pl.ds(0, n)],
                rows_v.at[pl.ds(buf * _L + off, n)],
                sems[buf],
            ).wait()

    def acc_store(i, buf):
        zero = jnp.zeros((16,), jnp.float32)
        rbase = buf * _L

        def acc_fn(r, carry):
            a0, a1, a2, a3 = carry
            a0 = a0 + rows_v[rbase + r, pl.ds(0, 16)]
            a1 = a1 + rows_v[rbase + r, pl.ds(16, 16)]
            a2 = a2 + rows_v[rbase + r, pl.ds(32, 16)]
            a3 = a3 + rows_v[rbase + r, pl.ds(48, 16)]
            return a0, a1, a2, a3

        a0, a1, a2, a3 = lax.fori_loop(
            0, _L, acc_fn, (zero, zero, zero, zero), unroll=2
        )
        scale = jnp.float32(1.0 / _L)
        out_v[pl.ds(0, 16)] = a0 * scale
        out_v[pl.ds(16, 16)] = a1 * scale
        out_v[pl.ds(32, 16)] = a2 * scale
        out_v[pl.ds(48, 16)] = a3 * scale
        pltpu.sync_copy(out_v, out_hbm.at[base + i])

    start_idx(base, 0)
    start_idx(base + 1, 1)
    wait_idx(0)
    start(0)

    def pair_fn(k, _):
        i0 = 2 * k
        wait_idx(1)
        start(1)
        drain(0)
        start_idx(base + jnp.minimum(i0 + 2, _BPW - 1), 0)
        acc_store(i0, 0)
        wait_idx(0)
        start(0)
        drain(1)
        start_idx(base + jnp.minimum(i0 + 3, _BPW - 1), 1)
        acc_store(i0 + 1, 1)
        return ()

    lax.fori_loop(0, _BPW // 2, pair_fn, ())
    wait_idx(1)
    drain(0)


@functools.partial(jax.jit, static_argnames=())
def _pool_mean(x2, table):
    mesh = plsc.VectorSubcoreMesh(core_axis_name="c", subcore_axis_name="s")
    k = pl.kernel(
        _pool_body,
        out_type=jax.ShapeDtypeStruct((_B, _EMB), jnp.float32),
        mesh=mesh,
        scratch_types=[
            pltpu.VMEM((2 * _L,), jnp.int32),
            pltpu.VMEM((2 * _L, _EMB), jnp.float32),
            pltpu.VMEM((_EMB,), jnp.float32),
            pltpu.SemaphoreType.DMA,
            pltpu.SemaphoreType.DMA,
            pltpu.SemaphoreType.DMA,
            pltpu.SemaphoreType.DMA,
        ],
        compiler_params=pltpu.CompilerParams(use_tc_tiling_on_sc=False),
    )
    return k(x2, table)


_VB = 4096
_VGRID = -(-_VOCAB // _VB)
_VPAD = _VGRID * _VB


def _detile_body(tT_ref, out_ref):
    blk = tT_ref[...]
    left = blk[:, : _VB // 2].T
    right = blk[:, _VB // 2 :].T
    out_ref[...] = jnp.concatenate([left, right], axis=1)


def _detile(tT):
    return pl.pallas_call(
        _detile_body,
        grid=(_VGRID,),
        in_specs=[pl.BlockSpec((_EMB, _VB), lambda i: (0, i))],
        out_specs=pl.BlockSpec((_VB // 2, 2 * _EMB), lambda i: (i, 0)),
        out_shape=jax.ShapeDtypeStruct((_VPAD // 2, 2 * _EMB), jnp.float32),
    )(tT)


def _mlp_body(h_ref, g1_ref, b1_ref, W1_ref, bf1_ref, g2_ref, b2_ref,
              W2_ref, bf2_ref, out_ref):
    h = h_ref[...]
    mu1 = jnp.mean(h, axis=0, keepdims=True)
    d1 = h - mu1
    var1 = jnp.mean(d1 * d1, axis=0, keepdims=True)
    h = d1 * lax.rsqrt(var1 + _EPS) * g1_ref[...] + b1_ref[...]
    h = lax.dot_general(h, W1_ref[...], (((1,), (1,)), ((), ())),
                        preferred_element_type=jnp.float32) + bf1_ref[...]
    mu2 = jnp.mean(h, axis=0, keepdims=True)
    d2 = h - mu2
    var2 = jnp.mean(d2 * d2, axis=0, keepdims=True)
    h = d2 * lax.rsqrt(var2 + _EPS) * g2_ref[...] + b2_ref[...]
    out = lax.dot_general(h, W2_ref[...], (((1,), (1,)), ((), ())),
                          preferred_element_type=jnp.float32) + bf2_ref[...]
    out_ref[...] = out


def _mlp(h, g1, b1, W1, bf1, g2, b2, W2, bf2):
    return pl.pallas_call(
        _mlp_body,
        out_shape=jax.ShapeDtypeStruct((_B, _NCLS), jnp.float32),
    )(h, g1.reshape(1, _EMB), b1.reshape(1, _EMB), W1, bf1.reshape(1, _DIM),
      g2.reshape(1, _DIM), b2.reshape(1, _DIM), W2, bf2.reshape(1, _NCLS))


def kernel(x, table, g1, b1, W1, bf1, g2, b2, W2, bf2):
    table_lin = _detile(table.T).reshape(_VPAD, _EMB)
    xr = (x & ~0xFFF) | ((x & 0x7FF) << 1) | ((x >> 11) & 1)
    h = _pool_mean(xr, table_lin)
    return _mlp(h, g1, b1, W1, bf1, g2, b2, W2, bf2)
